# baseline (speedup 1.0000x reference)
import jax
import jax.numpy as jnp
from jax.experimental import pallas as pl
from jax.experimental.pallas import tpu as pltpu

_VB = 3840


def _lm_head_kernel(ids_ref, emb_hbm, w_ref, out_ref, x_ref, sem):
    nb = x_ref.shape[0]

    @pl.when(pl.program_id(0) == 0)
    def _gather():
        for b in range(nb):
            pltpu.make_async_copy(
                emb_hbm.at[pl.ds(ids_ref[b], 1), :],
                x_ref.at[pl.ds(b, 1), :],
                sem,
            ).start()
        for b in range(nb):
            pltpu.make_async_copy(
                emb_hbm.at[pl.ds(ids_ref[b], 1), :],
                x_ref.at[pl.ds(b, 1), :],
                sem,
            ).wait()

    res = jax.lax.dot_general(
        x_ref[...],
        w_ref[...],
        dimension_numbers=(((1,), (1,)), ((), ())),
        preferred_element_type=jnp.float32,
    )
    out_ref[...] = res[:, None, :]


def kernel(input_ids, emb_table, W_out):
    B, S = input_ids.shape
    V, D = W_out.shape
    ids = input_ids.reshape(-1).astype(jnp.int32)
    nv = pl.cdiv(V, _VB)
    out = pl.pallas_call(
        _lm_head_kernel,
        grid_spec=pltpu.PrefetchScalarGridSpec(
            num_scalar_prefetch=1,
            grid=(nv,),
            in_specs=[
                pl.BlockSpec(memory_space=pltpu.MemorySpace.HBM),
                pl.BlockSpec((_VB, D), lambda v, ids: (v, 0)),
            ],
            out_specs=pl.BlockSpec((B, S, _VB), lambda v, ids: (0, 0, v)),
            scratch_shapes=[
                pltpu.VMEM((B * S, D), jnp.float32),
                pltpu.SemaphoreType.DMA,
            ],
        ),
        out_shape=jax.ShapeDtypeStruct((B, S, V), jnp.float32),
        compiler_params=pltpu.CompilerParams(
            dimension_semantics=("arbitrary",),
        ),
    )(ids, emb_table, W_out)
    return out


# speedup vs baseline: 1.0229x; 1.0229x over previous
import jax
import jax.numpy as jnp
from jax.experimental import pallas as pl
from jax.experimental.pallas import tpu as pltpu

_VB = 3456


def _lm_head_kernel(ids_ref, emb_hbm, w_ref, out_ref, x_ref, sem):
    nb = x_ref.shape[0]

    @pl.when(pl.program_id(0) == 0)
    def _gather():
        for b in range(nb):
            pltpu.make_async_copy(
                emb_hbm.at[pl.ds(ids_ref[b], 1), :],
                x_ref.at[pl.ds(b, 1), :],
                sem,
            ).start()
        for b in range(nb):
            pltpu.make_async_copy(
                emb_hbm.at[pl.ds(ids_ref[b], 1), :],
                x_ref.at[pl.ds(b, 1), :],
                sem,
            ).wait()

    res = jax.lax.dot_general(
        x_ref[...],
        w_ref[...],
        dimension_numbers=(((1,), (1,)), ((), ())),
        preferred_element_type=jnp.float32,
    )
    out_ref[...] = res[:, None, :]


def kernel(input_ids, emb_table, W_out):
    B, S = input_ids.shape
    V, D = W_out.shape
    ids = input_ids.reshape(-1).astype(jnp.int32)
    nv = pl.cdiv(V, _VB)
    out = pl.pallas_call(
        _lm_head_kernel,
        grid_spec=pltpu.PrefetchScalarGridSpec(
            num_scalar_prefetch=1,
            grid=(nv,),
            in_specs=[
                pl.BlockSpec(memory_space=pltpu.MemorySpace.HBM),
                pl.BlockSpec((_VB, D), lambda v, ids: (v, 0)),
            ],
            out_specs=pl.BlockSpec((B, S, _VB), lambda v, ids: (0, 0, v)),
            scratch_shapes=[
                pltpu.VMEM((B * S, D), jnp.float32),
                pltpu.SemaphoreType.DMA,
            ],
        ),
        out_shape=jax.ShapeDtypeStruct((B, S, V), jnp.float32),
        compiler_params=pltpu.CompilerParams(
            dimension_semantics=("arbitrary",),
        ),
    )(ids, emb_table, W_out)
    return out
